# baseline (speedup 1.0000x reference)
import functools
import math

import jax
import jax.numpy as jnp
from jax import lax
from jax.experimental import pallas as pl
from jax.experimental.pallas import tpu as pltpu
from jax.experimental.pallas import tpu_sc as plsc

NC = 2
NS = 16
NW = NC * NS
BPW = 512
K = 256
GCH = 128
LANES = 16


@functools.partial(jax.jit, static_argnums=(2, 3, 4))
def _emb_lookup(xt, table, bsz, seq, d_model):
    scale = d_model ** 0.5
    mesh = plsc.VectorSubcoreMesh(core_axis_name="c", subcore_axis_name="s")

    @functools.partial(
        pl.kernel,
        mesh=mesh,
        out_type=jax.ShapeDtypeStruct((seq, d_model, bsz), jnp.float32),
        scratch_types=[
            pltpu.VMEM((seq, BPW), jnp.int32),
            [pltpu.VMEM((K, d_model), jnp.float32)] * 2,
            [pltpu.VMEM((d_model, K), jnp.float32)] * 2,
            [pltpu.SemaphoreType.DMA] * 2,
            [pltpu.SemaphoreType.DMA] * 2,
        ],
        compiler_params=pltpu.CompilerParams(
            use_tc_tiling_on_sc=False, needs_layout_passes=False
        ),
    )
    def emb_kernel(xt_hbm, table_hbm, out_hbm, idx_v, rows, tbufs, gsems, wsems):
        wid = lax.axis_index("s") * NC + lax.axis_index("c")
        b0 = wid * BPW
        pltpu.sync_copy(xt_hbm.at[:, pl.ds(b0, BPW)], idx_v)

        iota = lax.iota(jnp.int32, LANES)
        fidx = [iota + j * LANES for j in range(d_model // LANES)]

        def gather(s, h, b, k):
            return pltpu.make_async_copy(
                table_hbm.at[idx_v.at[s, pl.ds(h * K + k * GCH, GCH)]],
                rows[b].at[pl.ds(k * GCH, GCH)],
                gsems[b],
            )

        def writeback(s, h, b):
            return pltpu.make_async_copy(
                tbufs[b], out_hbm.at[s, :, pl.ds(b0 + h * K, K)], wsems[b]
            )

        def outer(s, carry):
            for b in range(2):
                @pl.when(s > 0)
                def _drain():
                    writeback(s - 1, b, b).wait()

                for k in range(K // GCH):
                    gather(s, b, b, k).start()

            for b in range(2):
                for k in range(K // GCH):
                    gather(s, b, b, k).wait()

                def trans_row(i, carry2):
                    iidx = jnp.full((LANES,), 0, jnp.int32) + i
                    for j in range(d_model // LANES):
                        vec = rows[b][i, pl.ds(j * LANES, LANES)] * scale
                        plsc.store_scatter(tbufs[b], [fidx[j], iidx], vec)
                    return carry2

                lax.fori_loop(0, K, trans_row, 0, unroll=4)
                writeback(s, b, b).start()
            return carry

        lax.fori_loop(0, seq, outer, 0)
        for b in range(2):
            writeback(seq - 1, b, b).wait()

    return emb_kernel(xt, table)


def kernel(x, table):
    bsz, seq = x.shape
    vocab, d_model = table.shape
    assert bsz % (NW * BPW // BPW) == 0 and bsz // NW == BPW
    out3 = _emb_lookup(x.T.astype(jnp.int32), table, bsz, seq, d_model)
    return jnp.transpose(out3, (2, 0, 1))


# speedup vs baseline: 1.2457x; 1.2457x over previous
import functools
import math

import jax
import jax.numpy as jnp
from jax import lax
from jax.experimental import pallas as pl
from jax.experimental.pallas import tpu as pltpu
from jax.experimental.pallas import tpu_sc as plsc

NC = 2
NS = 16
NW = NC * NS
BPW = 512
K = 256
GCH = 128
LANES = 16


@functools.partial(jax.jit, static_argnums=(2, 3, 4))
def _emb_lookup(xt, table, bsz, seq, d_model):
    scale = d_model ** 0.5
    mesh = plsc.VectorSubcoreMesh(core_axis_name="c", subcore_axis_name="s")

    @functools.partial(
        pl.kernel,
        mesh=mesh,
        out_type=jax.ShapeDtypeStruct((seq, d_model, bsz), jnp.float32),
        scratch_types=[
            pltpu.VMEM((seq, BPW), jnp.int32),
            [pltpu.VMEM((K, d_model), jnp.float32)] * 2,
            [pltpu.VMEM((d_model, K), jnp.float32)] * 2,
            [pltpu.SemaphoreType.DMA] * 2,
            [pltpu.SemaphoreType.DMA] * 2,
        ],
        compiler_params=pltpu.CompilerParams(
            use_tc_tiling_on_sc=False, needs_layout_passes=False
        ),
    )
    def emb_kernel(xt_hbm, table_hbm, out_hbm, idx_v, rows, tbufs, gsems, wsems):
        wid = lax.axis_index("s") * NC + lax.axis_index("c")
        b0 = wid * BPW
        pltpu.sync_copy(xt_hbm.at[:, pl.ds(b0, BPW)], idx_v)

        iota = lax.iota(jnp.int32, LANES)
        fidx = [iota + j * LANES for j in range(d_model // LANES)]

        def gather(s, h, b, k):
            return pltpu.make_async_copy(
                table_hbm.at[idx_v.at[s, pl.ds(h * K + k * GCH, GCH)]],
                rows[b].at[pl.ds(k * GCH, GCH)],
                gsems[b],
            )

        def writeback(s, h, b):
            return pltpu.make_async_copy(
                tbufs[b], out_hbm.at[s, :, pl.ds(b0 + h * K, K)], wsems[b]
            )

        def outer(s, carry):
            for b in range(2):
                @pl.when(s > 0)
                def _drain():
                    writeback(s - 1, b, b).wait()

                for k in range(K // GCH):
                    gather(s, b, b, k).start()

            for b in range(2):
                for k in range(K // GCH):
                    gather(s, b, b, k).wait()

                @plsc.parallel_loop(0, K, unroll=8)
                def trans_row(i):
                    iidx = jnp.full((LANES,), 0, jnp.int32) + i
                    for j in range(d_model // LANES):
                        vec = rows[b][i, pl.ds(j * LANES, LANES)] * scale
                        plsc.store_scatter(tbufs[b], [fidx[j], iidx], vec)
                writeback(s, b, b).start()
            return carry

        lax.fori_loop(0, seq, outer, 0)
        for b in range(2):
            writeback(seq - 1, b, b).wait()

    return emb_kernel(xt, table)


def kernel(x, table):
    bsz, seq = x.shape
    vocab, d_model = table.shape
    assert bsz % (NW * BPW // BPW) == 0 and bsz // NW == BPW
    out3 = _emb_lookup(x.T.astype(jnp.int32), table, bsz, seq, d_model)
    return jnp.transpose(out3, (2, 0, 1))


# speedup vs baseline: 1.9368x; 1.5548x over previous
import functools

import jax
import jax.numpy as jnp
from jax import lax
from jax.experimental import pallas as pl
from jax.experimental.pallas import tpu as pltpu
from jax.experimental.pallas import tpu_sc as plsc

NC = 2
NS = 16
NW = NC * NS
BPW = 512
K = 128
KP = K + 1
NBUF = 4
LANES = 16


@functools.partial(jax.jit, static_argnums=(2, 3, 4))
def _emb_lookup(xt, table, bsz, seq, d_model):
    scale = d_model ** 0.5
    nh = BPW // K
    mesh = plsc.VectorSubcoreMesh(core_axis_name="c", subcore_axis_name="s")

    @functools.partial(
        pl.kernel,
        mesh=mesh,
        out_type=jax.ShapeDtypeStruct((seq, d_model, bsz), jnp.float32),
        scratch_types=[
            pltpu.VMEM((seq, BPW), jnp.int32),
            [pltpu.VMEM((K, d_model), jnp.float32)] * NBUF,
            [pltpu.VMEM((d_model, KP), jnp.float32)] * NBUF,
            [pltpu.SemaphoreType.DMA] * NBUF,
            [pltpu.SemaphoreType.DMA] * NBUF,
        ],
        compiler_params=pltpu.CompilerParams(
            use_tc_tiling_on_sc=False, needs_layout_passes=False
        ),
    )
    def emb_kernel(xt_hbm, table_hbm, out_hbm, idx_v, rows, tbufs, gsems, wsems):
        wid = lax.axis_index("s") * NC + lax.axis_index("c")
        b0 = wid * BPW
        pltpu.sync_copy(xt_hbm.at[:, pl.ds(b0, BPW)], idx_v)

        iota = lax.iota(jnp.int32, LANES)
        fidx = [iota + j * LANES for j in range(d_model // LANES)]

        def gather(s, h, b):
            return pltpu.make_async_copy(
                table_hbm.at[idx_v.at[s, pl.ds(h * K, K)]], rows[b], gsems[b]
            )

        def writeback(s, h, b):
            return pltpu.make_async_copy(
                tbufs[b].at[:, pl.ds(0, K)],
                out_hbm.at[s, :, pl.ds(b0 + h * K, K)],
                wsems[b],
            )

        def outer(s, carry):
            for h in range(nh):
                @pl.when(s > 0)
                def _drain():
                    writeback(s - 1, h, h).wait()

                gather(s, h, h).start()

            for h in range(nh):
                gather(s, h, h).wait()

                @plsc.parallel_loop(0, K, unroll=8)
                def trans_row(i):
                    iidx = jnp.full((LANES,), 0, jnp.int32) + i
                    for j in range(d_model // LANES):
                        vec = rows[h][i, pl.ds(j * LANES, LANES)] * scale
                        plsc.store_scatter(tbufs[h], [fidx[j], iidx], vec)

                writeback(s, h, h).start()
            return carry

        lax.fori_loop(0, seq, outer, 0)
        for h in range(nh):
            writeback(seq - 1, h, h).wait()

    return emb_kernel(xt, table)


def kernel(x, table):
    bsz, seq = x.shape
    vocab, d_model = table.shape
    assert bsz == NW * BPW and BPW % K == 0 and BPW // K == NBUF
    out3 = _emb_lookup(x.T.astype(jnp.int32), table, bsz, seq, d_model)
    return jnp.transpose(out3, (2, 0, 1))
